# baseline (speedup 1.0000x reference)
import functools

import jax
import jax.numpy as jnp
from jax import lax
from jax.experimental import pallas as pl
from jax.experimental.pallas import tpu as pltpu
from jax.experimental.pallas import tpu_sc as plsc

_B = 8
_H = 512
_W = 512
_N = _H * _W
_TOTAL = _B * _N
_NC = 2
_NS = 16
_NW = _NC * _NS
_T_TC = 352
_SC_ROWS = _H - _T_TC
_ROWS_W = _SC_ROWS // (_NW // _B)
_CHUNK_R = 24
_CHUNK_PLAN = []
_off = 0
while _off < _ROWS_W:
    _n = min(_CHUNK_R, _ROWS_W - _off)
    _CHUNK_PLAN.append((_off, _n))
    _off += _n
_UNROLL = 8
_NBANK = 4

_POSITIVE_MULT = 3.0
_POSITIVE_ID = 2
_NEGATIVE_ID = 1


def _sc_reduce_body(x_hbm, y_hbm, m_hbm, out_hbm,
                    xb0, yb0, mb0, xb1, yb1, mb1, ob, sem0, sem1):
    wid = lax.axis_index("s") * _NC + lax.axis_index("c")
    sample = wid // (_NW // _B)
    quarter = wid % (_NW // _B)
    row0 = _T_TC + quarter * _ROWS_W
    xbs, ybs, mbs = (xb0, xb1), (yb0, yb1), (mb0, mb1)
    sems = (sem0, sem1)

    def copies(ci, slot):
        off, n = _CHUNK_PLAN[ci]
        r = row0 + off
        return (
            pltpu.make_async_copy(
                x_hbm.at[sample, pl.ds(r, n)],
                xbs[slot].at[pl.ds(0, n)], sems[slot]),
            pltpu.make_async_copy(
                y_hbm.at[sample, pl.ds(r, n)],
                ybs[slot].at[pl.ds(0, n)], sems[slot]),
            pltpu.make_async_copy(
                m_hbm.at[sample, pl.ds(r, n)],
                mbs[slot].at[pl.ds(0, n)], sems[slot]),
        )

    zero = jnp.zeros((16,), jnp.float32)
    zi = jnp.zeros((16,), jnp.int32)
    acc = tuple([zero, zero, zi, zi] * _NBANK)
    vecs_per_row = _W // 16
    groups_per_row = vecs_per_row // _UNROLL

    def make_vec_body(slot):
        xb, yb, mb = xbs[slot], ybs[slot], mbs[slot]

        def vec_body(vi, a):
            banks = [list(a[4 * b:4 * b + 4]) for b in range(_NBANK)]
            row = vi // groups_per_row
            col = (vi % groups_per_row) * (16 * _UNROLL)
            for u in range(_UNROLL):
                sp, sn, cm, c2 = banks[u % _NBANK]
                oo = col + u * 16
                xv = xb[row, pl.ds(oo, 16)]
                yv = yb[row, pl.ds(oo, 16)]
                mv = mb[row, pl.ds(oo, 16)]
                d = xv - yv
                sl = d * d
                banks[u % _NBANK] = [
                    sp + jnp.where(mv == _POSITIVE_ID, sl, zero),
                    sn + jnp.where(mv == _NEGATIVE_ID, sl, zero),
                    cm + mv,
                    c2 + mv * mv,
                ]
            return tuple(v for b in banks for v in b)

        return vec_body

    for c in copies(0, 0):
        c.start()
    for ci in range(len(_CHUNK_PLAN)):
        slot = ci % 2
        if ci + 1 < len(_CHUNK_PLAN):
            for c in copies(ci + 1, 1 - slot):
                c.start()
        for c in copies(ci, slot):
            c.wait()
        nvec = _CHUNK_PLAN[ci][1] * (_W // 16) // _UNROLL
        acc = lax.fori_loop(0, nvec, make_vec_body(slot), acc)

    banks = [acc[4 * b:4 * b + 4] for b in range(_NBANK)]
    sp = sn = None
    cm = c2 = None
    for b in banks:
        sp = b[0] if sp is None else sp + b[0]
        sn = b[1] if sn is None else sn + b[1]
        cm = b[2] if cm is None else cm + b[2]
        c2 = b[3] if c2 is None else c2 + b[3]
    ob[0] = sp
    ob[1] = sn
    ob[2] = cm.astype(jnp.float32)
    ob[3] = c2.astype(jnp.float32)
    pltpu.sync_copy(ob, out_hbm.at[wid])


_sc_reduce = functools.partial(
    pl.kernel,
    mesh=plsc.VectorSubcoreMesh(core_axis_name="c", subcore_axis_name="s"),
    out_type=jax.ShapeDtypeStruct((_NW, 4, 16), jnp.float32),
    scratch_types=[
        pltpu.VMEM((_CHUNK_R, _W), jnp.float32),
        pltpu.VMEM((_CHUNK_R, _W), jnp.float32),
        pltpu.VMEM((_CHUNK_R, _W), jnp.int32),
        pltpu.VMEM((_CHUNK_R, _W), jnp.float32),
        pltpu.VMEM((_CHUNK_R, _W), jnp.float32),
        pltpu.VMEM((_CHUNK_R, _W), jnp.int32),
        pltpu.VMEM((4, 16), jnp.float32),
        pltpu.SemaphoreType.DMA,
        pltpu.SemaphoreType.DMA,
    ],
)(_sc_reduce_body)


_TCJ = 2
_TBLK = _T_TC // _TCJ


def _tc_reduce_body(x_ref, y_ref, m_ref, out_ref):
    xv = x_ref[0]
    yv = y_ref[0]
    mv = m_ref[0]
    d = xv - yv
    sl = d * d
    slp = jnp.where(mv == _POSITIVE_ID, sl, 0.0)
    sln = jnp.where(mv == _NEGATIVE_ID, sl, 0.0)
    sp = jnp.sum(slp, axis=0)
    sn = jnp.sum(sln, axis=0)
    cmf = jnp.sum(mv, axis=0).astype(jnp.float32)
    c2f = jnp.sum(mv * mv, axis=0).astype(jnp.float32)
    part = jnp.stack([sp, sn, cmf, c2f])[None]
    j = pl.program_id(1)

    @pl.when(j == 0)
    def _():
        out_ref[...] = part

    @pl.when(j > 0)
    def _():
        out_ref[...] += part


def _tc_reduce(x, y, m):
    return pl.pallas_call(
        _tc_reduce_body,
        grid=(_B, _TCJ),
        in_specs=[
            pl.BlockSpec((1, _TBLK, _W), lambda i, j: (i, j, 0)),
            pl.BlockSpec((1, _TBLK, _W), lambda i, j: (i, j, 0)),
            pl.BlockSpec((1, _TBLK, _W), lambda i, j: (i, j, 0)),
        ],
        out_specs=pl.BlockSpec((1, 4, _W), lambda i, j: (i, 0, 0)),
        out_shape=jax.ShapeDtypeStruct((_B, 4, _W), jnp.float32),
    )(x, y, m)


def _finalize_body(sc_ref, tc_ref, loss_ref, needs_ref, stats_ref):
    scp = sc_ref[...].reshape(_B, _NW // _B, 4, 16)
    stats = (jnp.sum(jnp.sum(scp, axis=3), axis=1)
             + jnp.sum(tc_ref[...], axis=2))
    sp = stats[:, 0:1]
    sn = stats[:, 1:2]
    cm = stats[:, 2:3]
    c2 = stats[:, 3:4]
    c_pos = (c2 - cm) * 0.5
    c_neg = 2.0 * cm - c2
    npos = c_pos.astype(jnp.int32)
    nneg = c_neg.astype(jnp.int32)
    k = (c_pos * _POSITIVE_MULT).astype(jnp.int32)
    cond = (k + npos >= npos + nneg) | (k <= 10)
    nz_loss = (sp + sn) / (c_pos + c_neg)
    loss_ref[...] = jnp.sum(nz_loss, axis=0, keepdims=True) * (1.0 / _B)
    needs_ref[...] = jnp.sum((~cond).astype(jnp.int32), axis=0, keepdims=True)
    stats_ref[...] = stats


def _finalize(sc_parts, tc_parts):
    return pl.pallas_call(
        _finalize_body,
        out_shape=(
            jax.ShapeDtypeStruct((1, 1), jnp.float32),
            jax.ShapeDtypeStruct((1, 1), jnp.int32),
            jax.ShapeDtypeStruct((_B, 4), jnp.float32),
        ),
    )(sc_parts, tc_parts)


def _topk_body(x_ref, y_ref, m_ref, out_ref):
    xv = x_ref[0]
    yv = y_ref[0]
    mv = m_ref[0]
    d = xv - yv
    sl = d * d
    bits = lax.bitcast_convert_type(sl, jnp.int32)
    negm = mv == _NEGATIVE_ID
    mbits = jnp.where(negm, bits, -1)
    npos = jnp.sum((mv == _POSITIVE_ID).astype(jnp.int32))
    k = (npos.astype(jnp.float32) * _POSITIVE_MULT).astype(jnp.int32)

    def bit_body(j, prefix):
        cand = prefix | (jnp.int32(1) << (30 - j))
        c = jnp.sum((mbits >= cand).astype(jnp.int32))
        return jnp.where(c >= k, cand, prefix)

    t = lax.fori_loop(0, 31, bit_body, jnp.int32(0))
    gt = mbits > t
    c_gt = jnp.sum(gt.astype(jnp.int32))
    s_gt = jnp.sum(jnp.where(gt, sl, 0.0))
    tval = lax.bitcast_convert_type(t, jnp.float32)
    nt_sum = s_gt + (k - c_gt).astype(jnp.float32) * tval
    out_ref[...] = jnp.full((1, 1, 128), nt_sum, dtype=jnp.float32)


def _topk_sums(x, y, m):
    out = pl.pallas_call(
        _topk_body,
        grid=(_B,),
        in_specs=[
            pl.BlockSpec((1, _H, _W), lambda i: (i, 0, 0)),
            pl.BlockSpec((1, _H, _W), lambda i: (i, 0, 0)),
            pl.BlockSpec((1, _H, _W), lambda i: (i, 0, 0)),
        ],
        out_specs=pl.BlockSpec((1, 1, 128), lambda i: (i, 0, 0)),
        out_shape=jax.ShapeDtypeStruct((_B, 1, 128), jnp.float32),
    )(x, y, m)
    return out[:, 0, 0]


def kernel(x, y, idmask):
    sc_parts = _sc_reduce(x, y, idmask)
    tc_parts = _tc_reduce(x, y, idmask)
    loss_nz, needs, stats = _finalize(sc_parts, tc_parts)

    def _heavy():
        s_pos, s_neg, cm, cm2 = (
            stats[:, 0], stats[:, 1], stats[:, 2], stats[:, 3])
        c_pos = (cm2 - cm) * 0.5
        c_neg = 2.0 * cm - cm2
        npos = c_pos.astype(jnp.int32)
        nneg = c_neg.astype(jnp.int32)
        k = (npos.astype(jnp.float32) * _POSITIVE_MULT).astype(jnp.int32)
        num_all = npos + nneg
        nz_loss = (s_pos + s_neg) / (c_pos + c_neg)
        pl_mean = s_pos / c_pos
        cond = (k + npos >= num_all) | (k <= 10)
        nt_mean = _topk_sums(x, y, idmask) / k.astype(jnp.float32)
        loss = jnp.where(cond, nz_loss, pl_mean + nt_mean)
        return jnp.mean(loss)

    return lax.cond(needs[0, 0] != 0, _heavy, lambda: loss_nz[0, 0])


# speedup vs baseline: 1.0733x; 1.0733x over previous
import functools

import jax
import jax.numpy as jnp
from jax import lax
from jax.experimental import pallas as pl
from jax.experimental.pallas import tpu as pltpu
from jax.experimental.pallas import tpu_sc as plsc

_B = 8
_H = 512
_W = 512
_N = _H * _W
_TOTAL = _B * _N
_NC = 2
_NS = 16
_NW = _NC * _NS
_T_TC = 352
_SC_ROWS = _H - _T_TC
_ROWS_W = _SC_ROWS // (_NW // _B)
_CHUNK_R = 24
_CHUNK_PLAN = []
_off = 0
while _off < _ROWS_W:
    _n = min(_CHUNK_R, _ROWS_W - _off)
    _CHUNK_PLAN.append((_off, _n))
    _off += _n
_UNROLL = 8
_NBANK = 4

_POSITIVE_MULT = 3.0
_POSITIVE_ID = 2
_NEGATIVE_ID = 1


def _sc_reduce_body(x_hbm, y_hbm, m_hbm, out_hbm,
                    xb0, yb0, mb0, xb1, yb1, mb1, ob, sem0, sem1):
    wid = lax.axis_index("s") * _NC + lax.axis_index("c")
    sample = wid // (_NW // _B)
    quarter = wid % (_NW // _B)
    row0 = _T_TC + quarter * _ROWS_W
    xbs, ybs, mbs = (xb0, xb1), (yb0, yb1), (mb0, mb1)
    sems = (sem0, sem1)

    def copies(ci, slot):
        off, n = _CHUNK_PLAN[ci]
        r = row0 + off
        return (
            pltpu.make_async_copy(
                x_hbm.at[sample, pl.ds(r, n)],
                xbs[slot].at[pl.ds(0, n)], sems[slot]),
            pltpu.make_async_copy(
                y_hbm.at[sample, pl.ds(r, n)],
                ybs[slot].at[pl.ds(0, n)], sems[slot]),
            pltpu.make_async_copy(
                m_hbm.at[sample, pl.ds(r, n)],
                mbs[slot].at[pl.ds(0, n)], sems[slot]),
        )

    zero = jnp.zeros((16,), jnp.float32)
    zi = jnp.zeros((16,), jnp.int32)
    acc = tuple([zero, zero, zi, zi] * _NBANK)
    vecs_per_row = _W // 16
    groups_per_row = vecs_per_row // _UNROLL

    def make_vec_body(slot):
        xb, yb, mb = xbs[slot], ybs[slot], mbs[slot]

        def vec_body(vi, a):
            banks = [list(a[4 * b:4 * b + 4]) for b in range(_NBANK)]
            row = vi // groups_per_row
            col = (vi % groups_per_row) * (16 * _UNROLL)
            for u in range(_UNROLL):
                sp, sn, cm, c2 = banks[u % _NBANK]
                oo = col + u * 16
                xv = xb[row, pl.ds(oo, 16)]
                yv = yb[row, pl.ds(oo, 16)]
                mv = mb[row, pl.ds(oo, 16)]
                d = xv - yv
                sl = d * d
                banks[u % _NBANK] = [
                    sp + jnp.where(mv == _POSITIVE_ID, sl, zero),
                    sn + jnp.where(mv == _NEGATIVE_ID, sl, zero),
                    cm + mv,
                    c2 + mv * mv,
                ]
            return tuple(v for b in banks for v in b)

        return vec_body

    for c in copies(0, 0):
        c.start()
    for ci in range(len(_CHUNK_PLAN)):
        slot = ci % 2
        if ci + 1 < len(_CHUNK_PLAN):
            for c in copies(ci + 1, 1 - slot):
                c.start()
        for c in copies(ci, slot):
            c.wait()
        nvec = _CHUNK_PLAN[ci][1] * (_W // 16) // _UNROLL
        acc = plsc.parallel_loop(0, nvec, 1, unroll=2, carry=acc)(
            make_vec_body(slot))

    banks = [acc[4 * b:4 * b + 4] for b in range(_NBANK)]
    sp = sn = None
    cm = c2 = None
    for b in banks:
        sp = b[0] if sp is None else sp + b[0]
        sn = b[1] if sn is None else sn + b[1]
        cm = b[2] if cm is None else cm + b[2]
        c2 = b[3] if c2 is None else c2 + b[3]
    ob[0] = sp
    ob[1] = sn
    ob[2] = cm.astype(jnp.float32)
    ob[3] = c2.astype(jnp.float32)
    pltpu.sync_copy(ob, out_hbm.at[wid])


_sc_reduce = functools.partial(
    pl.kernel,
    mesh=plsc.VectorSubcoreMesh(core_axis_name="c", subcore_axis_name="s"),
    out_type=jax.ShapeDtypeStruct((_NW, 4, 16), jnp.float32),
    scratch_types=[
        pltpu.VMEM((_CHUNK_R, _W), jnp.float32),
        pltpu.VMEM((_CHUNK_R, _W), jnp.float32),
        pltpu.VMEM((_CHUNK_R, _W), jnp.int32),
        pltpu.VMEM((_CHUNK_R, _W), jnp.float32),
        pltpu.VMEM((_CHUNK_R, _W), jnp.float32),
        pltpu.VMEM((_CHUNK_R, _W), jnp.int32),
        pltpu.VMEM((4, 16), jnp.float32),
        pltpu.SemaphoreType.DMA,
        pltpu.SemaphoreType.DMA,
    ],
)(_sc_reduce_body)


def _tc_reduce_body(x_ref, y_ref, m_ref, out_ref):
    xv = x_ref[0]
    yv = y_ref[0]
    mv = m_ref[0]
    d = xv - yv
    sl = d * d
    slp = jnp.where(mv == _POSITIVE_ID, sl, 0.0)
    sln = jnp.where(mv == _NEGATIVE_ID, sl, 0.0)
    sp = jnp.sum(slp, axis=0)
    sn = jnp.sum(sln, axis=0)
    cmf = jnp.sum(mv, axis=0).astype(jnp.float32)
    c2f = jnp.sum(mv * mv, axis=0).astype(jnp.float32)
    out_ref[...] = jnp.stack([sp, sn, cmf, c2f])[None]


def _tc_reduce(x, y, m):
    return pl.pallas_call(
        _tc_reduce_body,
        grid=(_B,),
        in_specs=[
            pl.BlockSpec((1, _T_TC, _W), lambda i: (i, 0, 0)),
            pl.BlockSpec((1, _T_TC, _W), lambda i: (i, 0, 0)),
            pl.BlockSpec((1, _T_TC, _W), lambda i: (i, 0, 0)),
        ],
        out_specs=pl.BlockSpec((1, 4, _W), lambda i: (i, 0, 0)),
        out_shape=jax.ShapeDtypeStruct((_B, 4, _W), jnp.float32),
    )(x, y, m)


def _finalize_body(sc_ref, tc_ref, loss_ref, needs_ref, stats_ref):
    scp = sc_ref[...].reshape(_B, _NW // _B, 4, 16)
    stats = (jnp.sum(jnp.sum(scp, axis=3), axis=1)
             + jnp.sum(tc_ref[...], axis=2))
    sp = stats[:, 0:1]
    sn = stats[:, 1:2]
    cm = stats[:, 2:3]
    c2 = stats[:, 3:4]
    c_pos = (c2 - cm) * 0.5
    c_neg = 2.0 * cm - c2
    npos = c_pos.astype(jnp.int32)
    nneg = c_neg.astype(jnp.int32)
    k = (c_pos * _POSITIVE_MULT).astype(jnp.int32)
    cond = (k + npos >= npos + nneg) | (k <= 10)
    nz_loss = (sp + sn) / (c_pos + c_neg)
    loss_ref[...] = jnp.sum(nz_loss, axis=0, keepdims=True) * (1.0 / _B)
    needs_ref[...] = jnp.sum((~cond).astype(jnp.int32), axis=0, keepdims=True)
    stats_ref[...] = stats


def _finalize(sc_parts, tc_parts):
    return pl.pallas_call(
        _finalize_body,
        out_shape=(
            jax.ShapeDtypeStruct((1, 1), jnp.float32),
            jax.ShapeDtypeStruct((1, 1), jnp.int32),
            jax.ShapeDtypeStruct((_B, 4), jnp.float32),
        ),
    )(sc_parts, tc_parts)


def _topk_body(x_ref, y_ref, m_ref, out_ref):
    xv = x_ref[0]
    yv = y_ref[0]
    mv = m_ref[0]
    d = xv - yv
    sl = d * d
    bits = lax.bitcast_convert_type(sl, jnp.int32)
    negm = mv == _NEGATIVE_ID
    mbits = jnp.where(negm, bits, -1)
    npos = jnp.sum((mv == _POSITIVE_ID).astype(jnp.int32))
    k = (npos.astype(jnp.float32) * _POSITIVE_MULT).astype(jnp.int32)

    def bit_body(j, prefix):
        cand = prefix | (jnp.int32(1) << (30 - j))
        c = jnp.sum((mbits >= cand).astype(jnp.int32))
        return jnp.where(c >= k, cand, prefix)

    t = lax.fori_loop(0, 31, bit_body, jnp.int32(0))
    gt = mbits > t
    c_gt = jnp.sum(gt.astype(jnp.int32))
    s_gt = jnp.sum(jnp.where(gt, sl, 0.0))
    tval = lax.bitcast_convert_type(t, jnp.float32)
    nt_sum = s_gt + (k - c_gt).astype(jnp.float32) * tval
    out_ref[...] = jnp.full((1, 1, 128), nt_sum, dtype=jnp.float32)


def _topk_sums(x, y, m):
    out = pl.pallas_call(
        _topk_body,
        grid=(_B,),
        in_specs=[
            pl.BlockSpec((1, _H, _W), lambda i: (i, 0, 0)),
            pl.BlockSpec((1, _H, _W), lambda i: (i, 0, 0)),
            pl.BlockSpec((1, _H, _W), lambda i: (i, 0, 0)),
        ],
        out_specs=pl.BlockSpec((1, 1, 128), lambda i: (i, 0, 0)),
        out_shape=jax.ShapeDtypeStruct((_B, 1, 128), jnp.float32),
    )(x, y, m)
    return out[:, 0, 0]


def kernel(x, y, idmask):
    sc_parts = _sc_reduce(x, y, idmask)
    tc_parts = _tc_reduce(x, y, idmask)
    loss_nz, needs, stats = _finalize(sc_parts, tc_parts)

    def _heavy():
        s_pos, s_neg, cm, cm2 = (
            stats[:, 0], stats[:, 1], stats[:, 2], stats[:, 3])
        c_pos = (cm2 - cm) * 0.5
        c_neg = 2.0 * cm - cm2
        npos = c_pos.astype(jnp.int32)
        nneg = c_neg.astype(jnp.int32)
        k = (npos.astype(jnp.float32) * _POSITIVE_MULT).astype(jnp.int32)
        num_all = npos + nneg
        nz_loss = (s_pos + s_neg) / (c_pos + c_neg)
        pl_mean = s_pos / c_pos
        cond = (k + npos >= num_all) | (k <= 10)
        nt_mean = _topk_sums(x, y, idmask) / k.astype(jnp.float32)
        loss = jnp.where(cond, nz_loss, pl_mean + nt_mean)
        return jnp.mean(loss)

    return lax.cond(needs[0, 0] != 0, _heavy, lambda: loss_nz[0, 0])


# speedup vs baseline: 1.0741x; 1.0007x over previous
import functools

import jax
import jax.numpy as jnp
from jax import lax
from jax.experimental import pallas as pl
from jax.experimental.pallas import tpu as pltpu
from jax.experimental.pallas import tpu_sc as plsc

_B = 8
_H = 512
_W = 512
_N = _H * _W
_TOTAL = _B * _N
_NC = 2
_NS = 16
_NW = _NC * _NS
_T_TC = 352
_SC_ROWS = _H - _T_TC
_ROWS_W = _SC_ROWS // (_NW // _B)
_CHUNK_R = 24
_CHUNK_PLAN = []
_off = 0
while _off < _ROWS_W:
    _n = min(_CHUNK_R, _ROWS_W - _off)
    _CHUNK_PLAN.append((_off, _n))
    _off += _n
_UNROLL = 8
_NBANK = 4

_POSITIVE_MULT = 3.0
_POSITIVE_ID = 2
_NEGATIVE_ID = 1


def _sc_reduce_body(x_hbm, y_hbm, m_hbm, out_hbm,
                    xb0, yb0, mb0, xb1, yb1, mb1, ob, sem0, sem1):
    wid = lax.axis_index("s") * _NC + lax.axis_index("c")
    sample = wid // (_NW // _B)
    quarter = wid % (_NW // _B)
    row0 = _T_TC + quarter * _ROWS_W
    xbs, ybs, mbs = (xb0, xb1), (yb0, yb1), (mb0, mb1)
    sems = (sem0, sem1)

    def copies(ci, slot):
        off, n = _CHUNK_PLAN[ci]
        r = row0 + off
        return (
            pltpu.make_async_copy(
                x_hbm.at[sample, pl.ds(r, n)],
                xbs[slot].at[pl.ds(0, n)], sems[slot]),
            pltpu.make_async_copy(
                y_hbm.at[sample, pl.ds(r, n)],
                ybs[slot].at[pl.ds(0, n)], sems[slot]),
            pltpu.make_async_copy(
                m_hbm.at[sample, pl.ds(r, n)],
                mbs[slot].at[pl.ds(0, n)], sems[slot]),
        )

    zero = jnp.zeros((16,), jnp.float32)
    zi = jnp.zeros((16,), jnp.int32)
    acc = tuple([zero, zero, zi, zi] * _NBANK)
    vecs_per_row = _W // 16
    groups_per_row = vecs_per_row // _UNROLL

    def make_vec_body(slot):
        xb, yb, mb = xbs[slot], ybs[slot], mbs[slot]

        def vec_body(vi, a):
            banks = [list(a[4 * b:4 * b + 4]) for b in range(_NBANK)]
            row = vi // groups_per_row
            col = (vi % groups_per_row) * (16 * _UNROLL)
            for u in range(_UNROLL):
                sp, sn, cm, c2 = banks[u % _NBANK]
                oo = col + u * 16
                xv = xb[row, pl.ds(oo, 16)]
                yv = yb[row, pl.ds(oo, 16)]
                mv = mb[row, pl.ds(oo, 16)]
                d = xv - yv
                sl = d * d
                banks[u % _NBANK] = [
                    sp + jnp.where(mv == _POSITIVE_ID, sl, zero),
                    sn + jnp.where(mv == _NEGATIVE_ID, sl, zero),
                    cm + mv,
                    c2 + mv * mv,
                ]
            return tuple(v for b in banks for v in b)

        return vec_body

    for c in copies(0, 0):
        c.start()
    for ci in range(len(_CHUNK_PLAN)):
        slot = ci % 2
        if ci + 1 < len(_CHUNK_PLAN):
            for c in copies(ci + 1, 1 - slot):
                c.start()
        for c in copies(ci, slot):
            c.wait()
        nvec = _CHUNK_PLAN[ci][1] * (_W // 16) // _UNROLL
        acc = plsc.parallel_loop(0, nvec, 1, unroll=2, carry=acc)(
            make_vec_body(slot))

    banks = [acc[4 * b:4 * b + 4] for b in range(_NBANK)]
    sp = sn = None
    cm = c2 = None
    for b in banks:
        sp = b[0] if sp is None else sp + b[0]
        sn = b[1] if sn is None else sn + b[1]
        cm = b[2] if cm is None else cm + b[2]
        c2 = b[3] if c2 is None else c2 + b[3]
    ob[0] = sp
    ob[1] = sn
    ob[2] = cm.astype(jnp.float32)
    ob[3] = c2.astype(jnp.float32)
    pltpu.sync_copy(ob, out_hbm.at[wid])


@functools.lru_cache(maxsize=None)
def _sc_reduce_kernel():
    return functools.partial(
        pl.kernel,
        mesh=plsc.VectorSubcoreMesh(core_axis_name="c", subcore_axis_name="s"),
        out_type=jax.ShapeDtypeStruct((_NW, 4, 16), jnp.float32),
        scratch_types=[
            pltpu.VMEM((_CHUNK_R, _W), jnp.float32),
            pltpu.VMEM((_CHUNK_R, _W), jnp.float32),
            pltpu.VMEM((_CHUNK_R, _W), jnp.int32),
            pltpu.VMEM((_CHUNK_R, _W), jnp.float32),
            pltpu.VMEM((_CHUNK_R, _W), jnp.float32),
            pltpu.VMEM((_CHUNK_R, _W), jnp.int32),
            pltpu.VMEM((4, 16), jnp.float32),
            pltpu.SemaphoreType.DMA,
            pltpu.SemaphoreType.DMA,
        ],
    )(_sc_reduce_body)


def _sc_reduce(x, y, m):
    return _sc_reduce_kernel()(x, y, m)


def _tc_reduce_body(x_ref, y_ref, m_ref, out_ref):
    xv = x_ref[0]
    yv = y_ref[0]
    mv = m_ref[0]
    d = xv - yv
    sl = d * d
    slp = jnp.where(mv == _POSITIVE_ID, sl, 0.0)
    sln = jnp.where(mv == _NEGATIVE_ID, sl, 0.0)
    sp = jnp.sum(slp, axis=0)
    sn = jnp.sum(sln, axis=0)
    cmf = jnp.sum(mv, axis=0).astype(jnp.float32)
    c2f = jnp.sum(mv * mv, axis=0).astype(jnp.float32)
    out_ref[...] = jnp.stack([sp, sn, cmf, c2f])[None]


def _tc_reduce(x, y, m):
    return pl.pallas_call(
        _tc_reduce_body,
        grid=(_B,),
        in_specs=[
            pl.BlockSpec((1, _T_TC, _W), lambda i: (i, 0, 0)),
            pl.BlockSpec((1, _T_TC, _W), lambda i: (i, 0, 0)),
            pl.BlockSpec((1, _T_TC, _W), lambda i: (i, 0, 0)),
        ],
        out_specs=pl.BlockSpec((1, 4, _W), lambda i: (i, 0, 0)),
        out_shape=jax.ShapeDtypeStruct((_B, 4, _W), jnp.float32),
    )(x, y, m)


def _finalize_body(sc_ref, tc_ref, loss_ref, needs_ref, stats_ref):
    scp = sc_ref[...].reshape(_B, _NW // _B, 4, 16)
    stats = (jnp.sum(jnp.sum(scp, axis=3), axis=1)
             + jnp.sum(tc_ref[...], axis=2))
    sp = stats[:, 0:1]
    sn = stats[:, 1:2]
    cm = stats[:, 2:3]
    c2 = stats[:, 3:4]
    c_pos = (c2 - cm) * 0.5
    c_neg = 2.0 * cm - c2
    npos = c_pos.astype(jnp.int32)
    nneg = c_neg.astype(jnp.int32)
    k = (c_pos * _POSITIVE_MULT).astype(jnp.int32)
    cond = (k + npos >= npos + nneg) | (k <= 10)
    nz_loss = (sp + sn) / (c_pos + c_neg)
    loss_ref[...] = jnp.sum(nz_loss, axis=0, keepdims=True) * (1.0 / _B)
    needs_ref[...] = jnp.sum((~cond).astype(jnp.int32), axis=0, keepdims=True)
    stats_ref[...] = stats


def _finalize(sc_parts, tc_parts):
    return pl.pallas_call(
        _finalize_body,
        out_shape=(
            jax.ShapeDtypeStruct((1, 1), jnp.float32),
            jax.ShapeDtypeStruct((1, 1), jnp.int32),
            jax.ShapeDtypeStruct((_B, 4), jnp.float32),
        ),
    )(sc_parts, tc_parts)


def _topk_body(x_ref, y_ref, m_ref, out_ref):
    xv = x_ref[0]
    yv = y_ref[0]
    mv = m_ref[0]
    d = xv - yv
    sl = d * d
    bits = lax.bitcast_convert_type(sl, jnp.int32)
    negm = mv == _NEGATIVE_ID
    mbits = jnp.where(negm, bits, -1)
    npos = jnp.sum((mv == _POSITIVE_ID).astype(jnp.int32))
    k = (npos.astype(jnp.float32) * _POSITIVE_MULT).astype(jnp.int32)

    def bit_body(j, prefix):
        cand = prefix | (jnp.int32(1) << (30 - j))
        c = jnp.sum((mbits >= cand).astype(jnp.int32))
        return jnp.where(c >= k, cand, prefix)

    t = lax.fori_loop(0, 31, bit_body, jnp.int32(0))
    gt = mbits > t
    c_gt = jnp.sum(gt.astype(jnp.int32))
    s_gt = jnp.sum(jnp.where(gt, sl, 0.0))
    tval = lax.bitcast_convert_type(t, jnp.float32)
    nt_sum = s_gt + (k - c_gt).astype(jnp.float32) * tval
    out_ref[...] = jnp.full((1, 1, 128), nt_sum, dtype=jnp.float32)


def _topk_sums(x, y, m):
    out = pl.pallas_call(
        _topk_body,
        grid=(_B,),
        in_specs=[
            pl.BlockSpec((1, _H, _W), lambda i: (i, 0, 0)),
            pl.BlockSpec((1, _H, _W), lambda i: (i, 0, 0)),
            pl.BlockSpec((1, _H, _W), lambda i: (i, 0, 0)),
        ],
        out_specs=pl.BlockSpec((1, 1, 128), lambda i: (i, 0, 0)),
        out_shape=jax.ShapeDtypeStruct((_B, 1, 128), jnp.float32),
    )(x, y, m)
    return out[:, 0, 0]


def kernel(x, y, idmask):
    sc_parts = _sc_reduce(x, y, idmask)
    tc_parts = _tc_reduce(x, y, idmask)
    loss_nz, needs, stats = _finalize(sc_parts, tc_parts)

    def _heavy():
        s_pos, s_neg, cm, cm2 = (
            stats[:, 0], stats[:, 1], stats[:, 2], stats[:, 3])
        c_pos = (cm2 - cm) * 0.5
        c_neg = 2.0 * cm - cm2
        npos = c_pos.astype(jnp.int32)
        nneg = c_neg.astype(jnp.int32)
        k = (npos.astype(jnp.float32) * _POSITIVE_MULT).astype(jnp.int32)
        num_all = npos + nneg
        nz_loss = (s_pos + s_neg) / (c_pos + c_neg)
        pl_mean = s_pos / c_pos
        cond = (k + npos >= num_all) | (k <= 10)
        nt_mean = _topk_sums(x, y, idmask) / k.astype(jnp.float32)
        loss = jnp.where(cond, nz_loss, pl_mean + nt_mean)
        return jnp.mean(loss)

    return lax.cond(needs[0, 0] != 0, _heavy, lambda: loss_nz[0, 0])


# speedup vs baseline: 1.1044x; 1.0282x over previous
import functools

import jax
import jax.numpy as jnp
from jax import lax
from jax.experimental import pallas as pl
from jax.experimental.pallas import tpu as pltpu
from jax.experimental.pallas import tpu_sc as plsc

_B = 8
_H = 512
_W = 512
_N = _H * _W
_TOTAL = _B * _N
_NC = 2
_NS = 16
_NW = _NC * _NS
_T_TC = 384
_SC_ROWS = _H - _T_TC
_ROWS_W = _SC_ROWS // (_NW // _B)
_CHUNK_R = 24
_CHUNK_PLAN = []
_off = 0
while _off < _ROWS_W:
    _n = min(_CHUNK_R, _ROWS_W - _off)
    _CHUNK_PLAN.append((_off, _n))
    _off += _n
_UNROLL = 8
_NBANK = 4

_POSITIVE_MULT = 3.0
_POSITIVE_ID = 2
_NEGATIVE_ID = 1


def _sc_reduce_body(x_hbm, y_hbm, m_hbm, out_hbm,
                    xb0, yb0, mb0, xb1, yb1, mb1, ob, sem0, sem1):
    wid = lax.axis_index("s") * _NC + lax.axis_index("c")
    sample = wid // (_NW // _B)
    quarter = wid % (_NW // _B)
    row0 = _T_TC + quarter * _ROWS_W
    xbs, ybs, mbs = (xb0, xb1), (yb0, yb1), (mb0, mb1)
    sems = (sem0, sem1)

    def copies(ci, slot):
        off, n = _CHUNK_PLAN[ci]
        r = row0 + off
        return (
            pltpu.make_async_copy(
                x_hbm.at[sample, pl.ds(r, n)],
                xbs[slot].at[pl.ds(0, n)], sems[slot]),
            pltpu.make_async_copy(
                y_hbm.at[sample, pl.ds(r, n)],
                ybs[slot].at[pl.ds(0, n)], sems[slot]),
            pltpu.make_async_copy(
                m_hbm.at[sample, pl.ds(r, n)],
                mbs[slot].at[pl.ds(0, n)], sems[slot]),
        )

    zero = jnp.zeros((16,), jnp.float32)
    zi = jnp.zeros((16,), jnp.int32)
    acc = tuple([zero, zero, zi, zi] * _NBANK)
    vecs_per_row = _W // 16
    groups_per_row = vecs_per_row // _UNROLL

    def make_vec_body(slot):
        xb, yb, mb = xbs[slot], ybs[slot], mbs[slot]

        def vec_body(vi, a):
            banks = [list(a[4 * b:4 * b + 4]) for b in range(_NBANK)]
            row = vi // groups_per_row
            col = (vi % groups_per_row) * (16 * _UNROLL)
            for u in range(_UNROLL):
                sp, sn, cm, c2 = banks[u % _NBANK]
                oo = col + u * 16
                xv = xb[row, pl.ds(oo, 16)]
                yv = yb[row, pl.ds(oo, 16)]
                mv = mb[row, pl.ds(oo, 16)]
                d = xv - yv
                sl = d * d
                banks[u % _NBANK] = [
                    sp + jnp.where(mv == _POSITIVE_ID, sl, zero),
                    sn + jnp.where(mv == _NEGATIVE_ID, sl, zero),
                    cm + mv,
                    c2 + mv * mv,
                ]
            return tuple(v for b in banks for v in b)

        return vec_body

    for c in copies(0, 0):
        c.start()
    for ci in range(len(_CHUNK_PLAN)):
        slot = ci % 2
        if ci + 1 < len(_CHUNK_PLAN):
            for c in copies(ci + 1, 1 - slot):
                c.start()
        for c in copies(ci, slot):
            c.wait()
        nvec = _CHUNK_PLAN[ci][1] * (_W // 16) // _UNROLL
        acc = plsc.parallel_loop(0, nvec, 1, unroll=2, carry=acc)(
            make_vec_body(slot))

    banks = [acc[4 * b:4 * b + 4] for b in range(_NBANK)]
    sp = sn = None
    cm = c2 = None
    for b in banks:
        sp = b[0] if sp is None else sp + b[0]
        sn = b[1] if sn is None else sn + b[1]
        cm = b[2] if cm is None else cm + b[2]
        c2 = b[3] if c2 is None else c2 + b[3]
    ob[0] = sp
    ob[1] = sn
    ob[2] = cm.astype(jnp.float32)
    ob[3] = c2.astype(jnp.float32)
    pltpu.sync_copy(ob, out_hbm.at[wid])


@functools.lru_cache(maxsize=None)
def _sc_reduce_kernel():
    return functools.partial(
        pl.kernel,
        mesh=plsc.VectorSubcoreMesh(core_axis_name="c", subcore_axis_name="s"),
        out_type=jax.ShapeDtypeStruct((_NW, 4, 16), jnp.float32),
        scratch_types=[
            pltpu.VMEM((_CHUNK_R, _W), jnp.float32),
            pltpu.VMEM((_CHUNK_R, _W), jnp.float32),
            pltpu.VMEM((_CHUNK_R, _W), jnp.int32),
            pltpu.VMEM((_CHUNK_R, _W), jnp.float32),
            pltpu.VMEM((_CHUNK_R, _W), jnp.float32),
            pltpu.VMEM((_CHUNK_R, _W), jnp.int32),
            pltpu.VMEM((4, 16), jnp.float32),
            pltpu.SemaphoreType.DMA,
            pltpu.SemaphoreType.DMA,
        ],
    )(_sc_reduce_body)


def _sc_reduce(x, y, m):
    return _sc_reduce_kernel()(x, y, m)


def _tc_reduce_body(x_ref, y_ref, m_ref, out_ref):
    xv = x_ref[0]
    yv = y_ref[0]
    mv = m_ref[0]
    d = xv - yv
    sl = d * d
    slp = jnp.where(mv == _POSITIVE_ID, sl, 0.0)
    sln = jnp.where(mv == _NEGATIVE_ID, sl, 0.0)
    sp = jnp.sum(slp, axis=0)
    sn = jnp.sum(sln, axis=0)
    cmf = jnp.sum(mv, axis=0).astype(jnp.float32)
    c2f = jnp.sum(mv * mv, axis=0).astype(jnp.float32)
    out_ref[...] = jnp.stack([sp, sn, cmf, c2f])[None]


def _tc_reduce(x, y, m):
    return pl.pallas_call(
        _tc_reduce_body,
        grid=(_B,),
        in_specs=[
            pl.BlockSpec((1, _T_TC, _W), lambda i: (i, 0, 0)),
            pl.BlockSpec((1, _T_TC, _W), lambda i: (i, 0, 0)),
            pl.BlockSpec((1, _T_TC, _W), lambda i: (i, 0, 0)),
        ],
        out_specs=pl.BlockSpec((1, 4, _W), lambda i: (i, 0, 0)),
        out_shape=jax.ShapeDtypeStruct((_B, 4, _W), jnp.float32),
    )(x, y, m)


def _finalize_body(sc_ref, tc_ref, loss_ref, needs_ref, stats_ref):
    scp = sc_ref[...].reshape(_B, _NW // _B, 4, 16)
    stats = (jnp.sum(jnp.sum(scp, axis=3), axis=1)
             + jnp.sum(tc_ref[...], axis=2))
    sp = stats[:, 0:1]
    sn = stats[:, 1:2]
    cm = stats[:, 2:3]
    c2 = stats[:, 3:4]
    c_pos = (c2 - cm) * 0.5
    c_neg = 2.0 * cm - c2
    npos = c_pos.astype(jnp.int32)
    nneg = c_neg.astype(jnp.int32)
    k = (c_pos * _POSITIVE_MULT).astype(jnp.int32)
    cond = (k + npos >= npos + nneg) | (k <= 10)
    nz_loss = (sp + sn) / (c_pos + c_neg)
    loss_ref[...] = jnp.sum(nz_loss, axis=0, keepdims=True) * (1.0 / _B)
    needs_ref[...] = jnp.sum((~cond).astype(jnp.int32), axis=0, keepdims=True)
    stats_ref[...] = stats


def _finalize(sc_parts, tc_parts):
    return pl.pallas_call(
        _finalize_body,
        out_shape=(
            jax.ShapeDtypeStruct((1, 1), jnp.float32),
            jax.ShapeDtypeStruct((1, 1), jnp.int32),
            jax.ShapeDtypeStruct((_B, 4), jnp.float32),
        ),
    )(sc_parts, tc_parts)


def _topk_body(x_ref, y_ref, m_ref, out_ref):
    xv = x_ref[0]
    yv = y_ref[0]
    mv = m_ref[0]
    d = xv - yv
    sl = d * d
    bits = lax.bitcast_convert_type(sl, jnp.int32)
    negm = mv == _NEGATIVE_ID
    mbits = jnp.where(negm, bits, -1)
    npos = jnp.sum((mv == _POSITIVE_ID).astype(jnp.int32))
    k = (npos.astype(jnp.float32) * _POSITIVE_MULT).astype(jnp.int32)

    def bit_body(j, prefix):
        cand = prefix | (jnp.int32(1) << (30 - j))
        c = jnp.sum((mbits >= cand).astype(jnp.int32))
        return jnp.where(c >= k, cand, prefix)

    t = lax.fori_loop(0, 31, bit_body, jnp.int32(0))
    gt = mbits > t
    c_gt = jnp.sum(gt.astype(jnp.int32))
    s_gt = jnp.sum(jnp.where(gt, sl, 0.0))
    tval = lax.bitcast_convert_type(t, jnp.float32)
    nt_sum = s_gt + (k - c_gt).astype(jnp.float32) * tval
    out_ref[...] = jnp.full((1, 1, 128), nt_sum, dtype=jnp.float32)


def _topk_sums(x, y, m):
    out = pl.pallas_call(
        _topk_body,
        grid=(_B,),
        in_specs=[
            pl.BlockSpec((1, _H, _W), lambda i: (i, 0, 0)),
            pl.BlockSpec((1, _H, _W), lambda i: (i, 0, 0)),
            pl.BlockSpec((1, _H, _W), lambda i: (i, 0, 0)),
        ],
        out_specs=pl.BlockSpec((1, 1, 128), lambda i: (i, 0, 0)),
        out_shape=jax.ShapeDtypeStruct((_B, 1, 128), jnp.float32),
    )(x, y, m)
    return out[:, 0, 0]


def kernel(x, y, idmask):
    sc_parts = _sc_reduce(x, y, idmask)
    tc_parts = _tc_reduce(x, y, idmask)
    loss_nz, needs, stats = _finalize(sc_parts, tc_parts)

    def _heavy():
        s_pos, s_neg, cm, cm2 = (
            stats[:, 0], stats[:, 1], stats[:, 2], stats[:, 3])
        c_pos = (cm2 - cm) * 0.5
        c_neg = 2.0 * cm - cm2
        npos = c_pos.astype(jnp.int32)
        nneg = c_neg.astype(jnp.int32)
        k = (npos.astype(jnp.float32) * _POSITIVE_MULT).astype(jnp.int32)
        num_all = npos + nneg
        nz_loss = (s_pos + s_neg) / (c_pos + c_neg)
        pl_mean = s_pos / c_pos
        cond = (k + npos >= num_all) | (k <= 10)
        nt_mean = _topk_sums(x, y, idmask) / k.astype(jnp.float32)
        loss = jnp.where(cond, nz_loss, pl_mean + nt_mean)
        return jnp.mean(loss)

    return lax.cond(needs[0, 0] != 0, _heavy, lambda: loss_nz[0, 0])
